# baseline (speedup 1.0000x reference)
import functools

import jax
import jax.numpy as jnp
from jax import lax
from jax.experimental import pallas as pl
from jax.experimental.pallas import tpu as pltpu
from jax.experimental.pallas import tpu_sc as plsc

NUM_FIELDS = 26
VOCAB = 100000
EMBED_DIM = 32
BATCH = 4096

_INFO = plsc.get_sparse_core_info()
_NC = _INFO.num_cores
_NS = _INFO.num_subcores
_NW = _NC * _NS
_TOTAL = BATCH * NUM_FIELDS
_PER_W = _TOTAL // _NW
_CHUNK = 128
_NCHUNK = _PER_W // _CHUNK
_LANES = 16


def _make_sc_gather():
    mesh = plsc.VectorSubcoreMesh(core_axis_name="c", subcore_axis_name="s")

    @functools.partial(
        pl.kernel,
        mesh=mesh,
        out_type=jax.ShapeDtypeStruct((_TOTAL, EMBED_DIM), jnp.float32),
        scratch_types=[
            pltpu.VMEM((_PER_W,), jnp.int32),
            pltpu.VMEM((_PER_W, EMBED_DIM), jnp.float32),
            pltpu.SemaphoreType.DMA,
        ],
        compiler_params=pltpu.CompilerParams(use_tc_tiling_on_sc=False),
    )
    def k(idx_hbm, tab_hbm, out_hbm, idx_v, rows_v, sem):
        wid = lax.axis_index("s") * _NC + lax.axis_index("c")
        base = wid * _PER_W

        pltpu.sync_copy(idx_hbm.at[pl.ds(base, _PER_W)], idx_v)

        lane = lax.iota(jnp.int32, _LANES)

        def add_off(t, _):
            pos = lane + t * _LANES
            off = (pos % NUM_FIELDS) * VOCAB
            sl = pl.ds(t * _LANES, _LANES)
            idx_v[sl] = idx_v[sl] + off
            return 0

        lax.fori_loop(0, _PER_W // _LANES, add_off, 0)

        def gather(j, _):
            src = tab_hbm.at[idx_v.at[pl.ds(j * _CHUNK, _CHUNK)]]
            dst = rows_v.at[pl.ds(j * _CHUNK, _CHUNK)]
            pltpu.async_copy(src, dst, sem).wait()
            return 0

        lax.fori_loop(0, _NCHUNK, gather, 0)

        pltpu.sync_copy(rows_v, out_hbm.at[pl.ds(base, _PER_W)])

    return k


_sc_gather = _make_sc_gather()


@jax.jit
def kernel(indices, tables):
    idx_flat = indices.astype(jnp.int32).reshape(_TOTAL)
    tab_flat = tables.reshape(NUM_FIELDS * VOCAB, EMBED_DIM)
    out = _sc_gather(idx_flat, tab_flat)
    return out.reshape(BATCH, NUM_FIELDS, EMBED_DIM)
